# Initial kernel scaffold; baseline (speedup 1.0000x reference)
import jax
import jax.numpy as jnp
from jax.experimental import pallas as pl


def kernel(input, table, W_ih, W_hh, b_ih, b_hh):
    raise NotImplementedError("write your pallas kernel here")


# speedup vs baseline: 1.9034x; 1.9034x over previous
import functools

import jax
import jax.numpy as jnp
from jax import lax
from jax.experimental import pallas as pl
from jax.experimental.pallas import tpu as pltpu
from jax.experimental.pallas import tpu_sc as plsc

WORD = 256
EMB = 16
H = WORD * EMB
T = 256
NB = H // T


def _make_sc_gather():
    info = plsc.get_sparse_core_info()
    nc, ns = info.num_cores, info.num_subcores
    nw = nc * ns
    n_active = 16
    rows_per = WORD // n_active
    mesh = plsc.VectorSubcoreMesh(core_axis_name="c", subcore_axis_name="s")

    @functools.partial(
        pl.kernel,
        mesh=mesh,
        compiler_params=pltpu.CompilerParams(needs_layout_passes=False),
        out_type=jax.ShapeDtypeStruct((WORD * EMB,), jnp.float32),
        scratch_types=[
            pltpu.VMEM((WORD * EMB,), jnp.float32),
            pltpu.VMEM((rows_per,), jnp.int32),
            pltpu.VMEM((rows_per * EMB,), jnp.float32),
        ],
    )
    def sc_gather(table_hbm, idx_hbm, out_hbm, table_v, idx_v, rows_v):
        wid = lax.axis_index("s") * nc + lax.axis_index("c")

        @pl.when(wid < n_active)
        def _():
            pltpu.sync_copy(table_hbm, table_v)
            pltpu.sync_copy(idx_hbm.at[pl.ds(wid * rows_per, rows_per)], idx_v)
            lanes = lax.iota(jnp.int32, 16)
            iv = idx_v[...]
            for k in range(rows_per):
                row = iv[k]
                vals = plsc.load_gather(table_v, [row * EMB + lanes])
                rows_v[pl.ds(k * EMB, EMB)] = vals
            pltpu.sync_copy(
                rows_v, out_hbm.at[pl.ds(wid * rows_per * EMB, rows_per * EMB)])

    return sc_gather


_sc_gather = _make_sc_gather()


def _lstm_body(x_ref, wi_ref, wg_ref, wo_ref,
               bi_ih, bg_ih, bo_ih, bi_hh, bg_hh, bo_hh,
               h_ref, c_ref):
    x = x_ref[...]
    dn = (((1,), (1,)), ((), ()))
    gi = lax.dot_general(x, wi_ref[...], dn, preferred_element_type=jnp.float32) \
        + bi_ih[...] + bi_hh[...]
    gg = lax.dot_general(x, wg_ref[...], dn, preferred_element_type=jnp.float32) \
        + bg_ih[...] + bg_hh[...]
    go = lax.dot_general(x, wo_ref[...], dn, preferred_element_type=jnp.float32) \
        + bo_ih[...] + bo_hh[...]
    i = jax.nn.sigmoid(gi)
    g = jnp.tanh(gg)
    o = jax.nn.sigmoid(go)
    c = i * g
    h_ref[...] = o * jnp.tanh(c)
    c_ref[...] = c


def _lstm_pallas(x, W_ih, b_ih2, b_hh2):
    w_spec = lambda off: pl.BlockSpec((T, H), lambda j, off=off: (j + off, 0))
    b_spec = lambda off: pl.BlockSpec((1, T), lambda j, off=off: (0, j + off))
    in_specs = [
        pl.BlockSpec((1, H), lambda j: (0, 0)),
        w_spec(0), w_spec(2 * NB), w_spec(3 * NB),
        b_spec(0), b_spec(2 * NB), b_spec(3 * NB),
        b_spec(0), b_spec(2 * NB), b_spec(3 * NB),
    ]
    out_specs = [pl.BlockSpec((1, T), lambda j: (0, j))] * 2
    out_shape = [jax.ShapeDtypeStruct((1, H), jnp.float32)] * 2
    return pl.pallas_call(
        _lstm_body,
        grid=(NB,),
        in_specs=in_specs,
        out_specs=out_specs,
        out_shape=out_shape,
    )(x, W_ih, W_ih, W_ih, b_ih2, b_ih2, b_ih2, b_hh2, b_hh2, b_hh2)


def kernel(input, table, W_ih, W_hh, b_ih, b_hh):
    del W_hh
    idx = input.astype(jnp.int32)
    emb = _sc_gather(table.reshape(WORD * EMB), idx)
    x = emb.reshape(1, H)
    h, c = _lstm_pallas(x, W_ih,
                        b_ih.reshape(1, 4 * H), b_hh.reshape(1, 4 * H))
    out = h.reshape(1, 1, H)
    return (out, out, c.reshape(1, 1, H))
